# baseline (speedup 1.0000x reference)
import functools

import jax
import jax.numpy as jnp
from jax import lax
from jax.experimental import pallas as pl
from jax.experimental.pallas import tpu as pltpu
from jax.experimental.pallas import tpu_sc as plsc

_F32 = jnp.float32
_BT = 256
_CC = 2048
_PB = 1024
_BIG = 2 ** 30


def _rowdot(a, b):
    return lax.dot_general(a.astype(jnp.bfloat16), b.astype(jnp.bfloat16),
                           (((1,), (1,)), ((), ())),
                           preferred_element_type=_F32)


def _norm_rows_asrow(m):
    return jnp.sum(m * m, axis=1, keepdims=True).T


def _encode_argmin_body(x_ref, w1_ref, b1_ref, w2_ref, b2_ref, lw_ref,
                        lb_ref, lci_ref, cb_ref, idx_ref):
    x = x_ref[:]
    h = jax.nn.gelu(_rowdot(x, w1_ref[:]) + b1_ref[:])
    h = jax.nn.gelu(_rowdot(h, w2_ref[:]) + b2_ref[:])
    lw = lw_ref[:]
    absrowsum = jnp.sum(jnp.abs(lw), axis=1, keepdims=True)
    scale = jnp.minimum(1.0, jax.nn.softplus(lci_ref[:]) / absrowsum)
    z_e = jax.nn.sigmoid(_rowdot(h, lw * scale) + lb_ref[:])
    z_norm = jnp.sum(z_e * z_e, axis=1, keepdims=True)

    n_codes = cb_ref.shape[0]
    ids = lax.broadcasted_iota(jnp.int32, (_BT, _CC), 1)
    best_m = jnp.full((_BT, 1), jnp.inf, _F32)
    best_i = jnp.zeros((_BT, 1), jnp.int32)
    for c in range(n_codes // _CC):
        cbc = cb_ref[c * _CC:(c + 1) * _CC, :]
        dist = (z_norm + _norm_rows_asrow(cbc)) - 2.0 * _rowdot(z_e, cbc)
        m = jnp.min(dist, axis=1, keepdims=True)
        a = jnp.min(jnp.where(dist == m, ids, _BIG), axis=1, keepdims=True)
        better = m < best_m
        best_i = jnp.where(better, a + c * _CC, best_i)
        best_m = jnp.where(better, m, best_m)
    idx_ref[:] = best_i


@functools.lru_cache(maxsize=None)
def _encode_argmin_call(B, F, H1, H2, Z, NZ):
    grid = (B // _BT,)
    full = lambda i: (0, 0)
    return pl.pallas_call(
        _encode_argmin_body,
        grid=grid,
        in_specs=[
            pl.BlockSpec((_BT, F), lambda i: (i, 0)),
            pl.BlockSpec((H1, F), full),
            pl.BlockSpec((1, H1), full),
            pl.BlockSpec((H2, H1), full),
            pl.BlockSpec((1, H2), full),
            pl.BlockSpec((Z, H2), full),
            pl.BlockSpec((1, Z), full),
            pl.BlockSpec((Z, 1), full),
            pl.BlockSpec((NZ, Z), full),
        ],
        out_specs=pl.BlockSpec((_BT, 1), lambda i: (i, 0)),
        out_shape=jax.ShapeDtypeStruct((B, 1), jnp.int32),
    )


def _prep_body(cb_ref, qw_ref, qb_ref, qci_ref, qcb_ref, out_ref):
    qw = qw_ref[:]
    absrowsum = jnp.sum(jnp.abs(qw), axis=1, keepdims=True)
    scale = jnp.minimum(1.0, jax.nn.softplus(qci_ref[:]) / absrowsum)
    q_e = jax.nn.sigmoid(_rowdot(cb_ref[:], qw * scale) + qb_ref[:])
    qe_norm = jnp.sum(q_e * q_e, axis=1, keepdims=True)
    qcb = qcb_ref[:]
    nq = qcb.shape[0]
    dist = (qe_norm + _norm_rows_asrow(qcb)) - 2.0 * _rowdot(q_e, qcb)
    m = jnp.min(dist, axis=1, keepdims=True)
    ids = lax.broadcasted_iota(jnp.int32, (_PB, nq), 1)
    qidx = jnp.min(jnp.where(dist == m, ids, _BIG), axis=1, keepdims=True)
    onehot = (ids == qidx).astype(_F32)
    qq = lax.dot_general(onehot, qcb, (((1,), (0,)), ((), ())),
                         preferred_element_type=_F32,
                         precision=lax.Precision.HIGHEST)
    idxf = lax.bitcast_convert_type(qidx, _F32)
    pad = 128 - cb_ref.shape[1] - qq.shape[1] - 8
    out_ref[:] = jnp.concatenate(
        [cb_ref[:], qq, jnp.broadcast_to(idxf, (_PB, 8)),
         jnp.zeros((_PB, pad), _F32)], axis=1)


@functools.lru_cache(maxsize=None)
def _prep_call(NZ, Z, Q, NQ):
    grid = (NZ // _PB,)
    full = lambda i: (0, 0)
    return pl.pallas_call(
        _prep_body,
        grid=grid,
        in_specs=[
            pl.BlockSpec((_PB, Z), lambda i: (i, 0)),
            pl.BlockSpec((Q, Z), full),
            pl.BlockSpec((1, Q), full),
            pl.BlockSpec((Q, 1), full),
            pl.BlockSpec((NQ, Q), full),
        ],
        out_specs=pl.BlockSpec((_PB, 128), lambda i: (i, 0)),
        out_shape=jax.ShapeDtypeStruct((NZ, 128), _F32),
    )


@functools.lru_cache(maxsize=None)
def _gather_call(B, TW):
    info = plsc.get_sparse_core_info()
    nc, ns = info.num_cores, info.num_subcores
    nw = nc * ns
    bpw = B // nw
    ic = 128
    k = bpw // ic
    mesh = plsc.VectorSubcoreMesh(core_axis_name="c", subcore_axis_name="s")

    @functools.partial(
        pl.kernel, mesh=mesh,
        out_type=jax.ShapeDtypeStruct((B, TW), _F32),
        scratch_types=[
            pltpu.VMEM((k, ic), jnp.int32),
            pltpu.VMEM((bpw, TW), _F32),
            pltpu.SemaphoreType.DMA,
        ],
    )
    def _sc_gather(t2_hbm, idx_hbm, gt_hbm, idx_v, rows_t, sem):
        wid = lax.axis_index("s") * nc + lax.axis_index("c")
        pltpu.sync_copy(idx_hbm.at[pl.ds(wid * k, k)], idx_v)
        copies = [
            pltpu.async_copy(t2_hbm.at[idx_v.at[j]],
                             rows_t.at[pl.ds(j * ic, ic)], sem)
            for j in range(k)
        ]
        for c in copies:
            c.wait()
        pltpu.sync_copy(rows_t, gt_hbm.at[pl.ds(wid * bpw, bpw)])

    return _sc_gather


def kernel(x, enc_W1, enc_b1, enc_W2, enc_b2, lat_W, lat_b, lat_ci,
           z_codebook, qenc_W, qenc_b, qenc_ci, q_codebook):
    B, F = x.shape
    H1 = enc_W1.shape[0]
    H2 = enc_W2.shape[0]
    Z = lat_W.shape[0]
    NZ = z_codebook.shape[0]
    NQ, Q = q_codebook.shape

    z_idx = _encode_argmin_call(B, F, H1, H2, Z, NZ)(
        x, enc_W1, enc_b1.reshape(1, H1), enc_W2, enc_b2.reshape(1, H2),
        lat_W, lat_b.reshape(1, Z), lat_ci.reshape(Z, 1),
        z_codebook).reshape(B)
    t2 = _prep_call(NZ, Z, Q, NQ)(
        z_codebook, qenc_W, qenc_b.reshape(1, Q), qenc_ci.reshape(Q, 1),
        q_codebook)
    gt = _gather_call(B, 128)(t2, z_idx.reshape(B // 128, 128))
    z_q = gt[:, :Z]
    q_q = gt[:, Z:Z + Q]
    q_idx = lax.bitcast_convert_type(gt[:, Z + Q], jnp.int32)
    return z_q, z_idx, q_q, q_idx


# speedup vs baseline: 1.0016x; 1.0016x over previous
import functools

import jax
import jax.numpy as jnp
from jax import lax
from jax.experimental import pallas as pl
from jax.experimental.pallas import tpu as pltpu
from jax.experimental.pallas import tpu_sc as plsc

_F32 = jnp.float32
_BT = 256
_CC = 2048
_PB = 1024
_BIG = 2 ** 30


def _rowdot(a, b):
    return lax.dot_general(a.astype(jnp.bfloat16), b.astype(jnp.bfloat16),
                           (((1,), (1,)), ((), ())),
                           preferred_element_type=_F32)


def _norm_rows_asrow(m):
    return jnp.sum(m * m, axis=1, keepdims=True).T


def _encode_argmin_body(x_ref, w1_ref, b1_ref, w2_ref, b2_ref, lw_ref,
                        lb_ref, lci_ref, cb_ref, idx_ref):
    x = x_ref[:]
    h = jax.nn.gelu(_rowdot(x, w1_ref[:]) + b1_ref[:])
    h = jax.nn.gelu(_rowdot(h, w2_ref[:]) + b2_ref[:])
    lw = lw_ref[:]
    absrowsum = jnp.sum(jnp.abs(lw), axis=1, keepdims=True)
    scale = jnp.minimum(1.0, jax.nn.softplus(lci_ref[:]) / absrowsum)
    z_e = jax.nn.sigmoid(_rowdot(h, lw * scale) + lb_ref[:])
    z_norm = jnp.sum(z_e * z_e, axis=1, keepdims=True)

    n_codes = cb_ref.shape[0]
    ids = lax.broadcasted_iota(jnp.int32, (_BT, _CC), 1)
    best_m = jnp.full((_BT, 1), jnp.inf, _F32)
    best_i = jnp.zeros((_BT, 1), jnp.int32)
    for c in range(n_codes // _CC):
        cbc = cb_ref[c * _CC:(c + 1) * _CC, :]
        dist = (z_norm + _norm_rows_asrow(cbc)) - 2.0 * _rowdot(z_e, cbc)
        m = jnp.min(dist, axis=1, keepdims=True)
        a = jnp.min(jnp.where(dist == m, ids, _BIG), axis=1, keepdims=True)
        better = m < best_m
        best_i = jnp.where(better, a + c * _CC, best_i)
        best_m = jnp.where(better, m, best_m)
    idx_ref[:] = best_i


@functools.lru_cache(maxsize=None)
def _encode_argmin_call(B, F, H1, H2, Z, NZ):
    grid = (B // _BT,)
    full = lambda i: (0, 0)
    return pl.pallas_call(
        _encode_argmin_body,
        grid=grid,
        in_specs=[
            pl.BlockSpec((_BT, F), lambda i: (i, 0)),
            pl.BlockSpec((H1, F), full),
            pl.BlockSpec((1, H1), full),
            pl.BlockSpec((H2, H1), full),
            pl.BlockSpec((1, H2), full),
            pl.BlockSpec((Z, H2), full),
            pl.BlockSpec((1, Z), full),
            pl.BlockSpec((Z, 1), full),
            pl.BlockSpec((NZ, Z), full),
        ],
        out_specs=pl.BlockSpec((_BT, 1), lambda i: (i, 0)),
        out_shape=jax.ShapeDtypeStruct((B, 1), jnp.int32),
    )


def _prep_body(cb_ref, qw_ref, qb_ref, qci_ref, qcb_ref, out_ref):
    qw = qw_ref[:]
    absrowsum = jnp.sum(jnp.abs(qw), axis=1, keepdims=True)
    scale = jnp.minimum(1.0, jax.nn.softplus(qci_ref[:]) / absrowsum)
    q_e = jax.nn.sigmoid(_rowdot(cb_ref[:], qw * scale) + qb_ref[:])
    qe_norm = jnp.sum(q_e * q_e, axis=1, keepdims=True)
    qcb = qcb_ref[:]
    nq = qcb.shape[0]
    dist = (qe_norm + _norm_rows_asrow(qcb)) - 2.0 * _rowdot(q_e, qcb)
    m = jnp.min(dist, axis=1, keepdims=True)
    ids = lax.broadcasted_iota(jnp.int32, (_PB, nq), 1)
    qidx = jnp.min(jnp.where(dist == m, ids, _BIG), axis=1, keepdims=True)
    onehot = (ids == qidx).astype(_F32)
    qq = lax.dot_general(onehot, qcb, (((1,), (0,)), ((), ())),
                         preferred_element_type=_F32,
                         precision=lax.Precision.HIGHEST)
    idxf = lax.bitcast_convert_type(qidx, _F32)
    pad = 128 - cb_ref.shape[1] - qq.shape[1] - 8
    out_ref[:] = jnp.concatenate(
        [cb_ref[:], qq, jnp.broadcast_to(idxf, (_PB, 8)),
         jnp.zeros((_PB, pad), _F32)], axis=1)


@functools.lru_cache(maxsize=None)
def _prep_call(NZ, Z, Q, NQ):
    grid = (NZ // _PB,)
    full = lambda i: (0, 0)
    return pl.pallas_call(
        _prep_body,
        grid=grid,
        in_specs=[
            pl.BlockSpec((_PB, Z), lambda i: (i, 0)),
            pl.BlockSpec((Q, Z), full),
            pl.BlockSpec((1, Q), full),
            pl.BlockSpec((Q, 1), full),
            pl.BlockSpec((NQ, Q), full),
        ],
        out_specs=pl.BlockSpec((_PB, 128), lambda i: (i, 0)),
        out_shape=jax.ShapeDtypeStruct((NZ, 128), _F32),
    )


@functools.lru_cache(maxsize=None)
def _gather_call(B, TW):
    info = plsc.get_sparse_core_info()
    nc, ns = info.num_cores, info.num_subcores
    nw = nc * ns
    bpw = B // nw
    ic = 64
    k = bpw // ic
    mesh = plsc.VectorSubcoreMesh(core_axis_name="c", subcore_axis_name="s")

    @functools.partial(
        pl.kernel, mesh=mesh,
        out_type=jax.ShapeDtypeStruct((B, TW), _F32),
        scratch_types=[
            pltpu.VMEM((k, ic), jnp.int32),
        ] + [pltpu.VMEM((ic, TW), _F32) for _ in range(k)] + [
            pltpu.SemaphoreType.DMA,
        ],
    )
    def _sc_gather(t2_hbm, idx_hbm, gt_hbm, idx_v, *bufs_sem):
        bufs, sem = bufs_sem[:k], bufs_sem[k]
        wid = lax.axis_index("s") * nc + lax.axis_index("c")
        with jax.named_scope("idx_load"):
            pltpu.sync_copy(idx_hbm.at[pl.ds(wid * k, k)], idx_v)
        with jax.named_scope("gather"):
            copies = [
                pltpu.async_copy(t2_hbm.at[idx_v.at[j]], bufs[j], sem)
                for j in range(k)
            ]
            for c in copies:
                c.wait()
        with jax.named_scope("writeback"):
            for j in range(k):
                pltpu.sync_copy(
                    bufs[j], gt_hbm.at[pl.ds(wid * bpw + j * ic, ic)])

    return _sc_gather


def kernel(x, enc_W1, enc_b1, enc_W2, enc_b2, lat_W, lat_b, lat_ci,
           z_codebook, qenc_W, qenc_b, qenc_ci, q_codebook):
    B, F = x.shape
    H1 = enc_W1.shape[0]
    H2 = enc_W2.shape[0]
    Z = lat_W.shape[0]
    NZ = z_codebook.shape[0]
    NQ, Q = q_codebook.shape

    z_idx = _encode_argmin_call(B, F, H1, H2, Z, NZ)(
        x, enc_W1, enc_b1.reshape(1, H1), enc_W2, enc_b2.reshape(1, H2),
        lat_W, lat_b.reshape(1, Z), lat_ci.reshape(Z, 1),
        z_codebook).reshape(B)
    t2 = _prep_call(NZ, Z, Q, NQ)(
        z_codebook, qenc_W, qenc_b.reshape(1, Q), qenc_ci.reshape(Q, 1),
        q_codebook)
    gt = _gather_call(B, 128)(t2, z_idx.reshape(B // 64, 64))
    z_q = gt[:, :Z]
    q_q = gt[:, Z:Z + Q]
    q_idx = lax.bitcast_convert_type(gt[:, Z + Q], jnp.int32)
    return z_q, z_idx, q_q, q_idx
